# baseline (speedup 1.0000x reference)
import jax
import jax.numpy as jnp
from jax import lax
from jax.experimental import pallas as pl
from jax.experimental.pallas import tpu as pltpu
from jax.experimental.pallas import tpu_sc as plsc

N, K, F, N_UP = 10000, 32, 128, 2500
NC, NS = 2, 16
NW = NC * NS
NP = 10240
ROWS_PER_TILE = NP // NW
B = 4
NBLK = ROWS_PER_TILE // B
CR = B * K
DST_PER_TILE = NP // NS
DR = NP // F
NUP_P = 2560
UP_PER_TILE = NUP_P // NW
NV = F // 16

_mesh = plsc.VectorSubcoreMesh(
    core_axis_name="c", subcore_axis_name="s", num_cores=NC, num_subcores=NS)


def _push_body(feat_hbm, nidxf_hbm, wf_hbm,
               out0_hbm, out1_hbm, den0_hbm, den1_hbm,
               feat_v, w_v, idx_v, idxs_v, contrib_v, den_v, idxid_v,
               acc_sh, den_sh,
               isem0, isem1, ssem0, ssem1, zsem):
    c = lax.axis_index("c")
    s = lax.axis_index("s")
    wid = c * NS + s
    zvec = jnp.zeros((16,), jnp.float32)
    isems = (isem0, isem1)
    ssems = (ssem0, ssem1)
    nblk = (jnp.minimum(ROWS_PER_TILE, N - wid * ROWS_PER_TILE)
            + (B - 1)) // B

    def in_copies(b, buf):
        base = wid * ROWS_PER_TILE + b * B
        return (
            pltpu.make_async_copy(feat_hbm.at[pl.ds(base, B)],
                                  feat_v.at[buf], isems[buf]),
            pltpu.make_async_copy(wf_hbm.at[pl.ds(base * K, CR)],
                                  w_v.at[buf], isems[buf]),
            pltpu.make_async_copy(nidxf_hbm.at[pl.ds(base * K, CR)],
                                  idx_v.at[buf], isems[buf]),
        )

    def fire_inputs(b, buf):
        for d in in_copies(b, buf):
            d.start()

    def drain_inputs(b, buf):
        for d in in_copies(b, buf):
            d.wait()

    fire_inputs(0, 0)

    def zrow(j, carry):
        for v in range(NV):
            contrib_v[1, j, pl.ds(v * 16, 16)] = zvec
        return carry
    lax.fori_loop(0, CR, zrow, 0)

    zcopies = [
        pltpu.make_async_copy(
            contrib_v.at[1],
            acc_sh.at[pl.ds(s * DST_PER_TILE + j * CR, CR)], zsem)
        for j in range(DST_PER_TILE // CR)
    ]
    zdcopy = pltpu.make_async_copy(contrib_v.at[1, pl.ds(0, 8)],
                                   den_sh.at[pl.ds(s * 8, 8)], zsem)
    for d in zcopies:
        d.start()

    @pl.when(s < DR // 8)
    def _():
        zdcopy.start()

    def zden(j, carry):
        for v in range(NV):
            den_v[j, pl.ds(v * 16, 16)] = zvec
        return carry
    lax.fori_loop(0, DR, zden, 0)

    iota16 = lax.iota(jnp.int32, 16)
    for g in range(DR // 16):
        idxid_v[pl.ds(g * 16, 16)] = iota16 + g * 16

    for d in zcopies:
        d.wait()

    @pl.when(s < DR // 8)
    def _():
        zdcopy.wait()
    plsc.subcore_barrier()

    def pair(p, carry):
        for par in range(2):
            b = 2 * p + par
            drain_inputs(b, par)

            @pl.when(p > 0)
            def _():
                pltpu.make_async_copy(
                    contrib_v.at[par], acc_sh.at[idxs_v.at[par]],
                    ssems[par]).wait()

            @pl.when(b + 1 < nblk)
            def _():
                fire_inputs(b + 1, 1 - par)

            for i in range(B):
                f = [feat_v[par, i, pl.ds(v * 16, 16)] for v in range(NV)]
                for g in range(K // 16):
                    wvec = w_v[par, pl.ds(i * K + g * 16, 16)]
                    for kk in range(16):
                        m = i * K + g * 16 + kk
                        wk = wvec[kk]
                        for v in range(NV):
                            contrib_v[par, m, pl.ds(v * 16, 16)] = wk * f[v]
            for g in range(CR // 16):
                ivec = idx_v[par, pl.ds(g * 16, 16)]
                wvec = w_v[par, pl.ds(g * 16, 16)]
                idxs_v[par, pl.ds(g * 16, 16)] = ivec
                plsc.addupdate_scatter(
                    den_v,
                    [lax.shift_right_logical(ivec, 7),
                     lax.bitwise_and(ivec, 127)],
                    wvec)
            pltpu.async_copy(contrib_v.at[par], acc_sh.at[idxs_v.at[par]],
                             ssems[par], add=True)
        return carry
    lax.fori_loop(0, nblk // 2, pair, 0)

    for par in range(2):
        pltpu.make_async_copy(
            contrib_v.at[par], acc_sh.at[idxs_v.at[par]], ssems[par]).wait()

    pltpu.sync_copy(den_v, den_sh.at[idxid_v], add=True)
    plsc.subcore_barrier()

    @pl.when(c == 0)
    def _():
        pltpu.sync_copy(acc_sh.at[pl.ds(s * DST_PER_TILE, DST_PER_TILE)],
                        out0_hbm.at[pl.ds(s * DST_PER_TILE, DST_PER_TILE)])

        @pl.when(s < DR // 8)
        def _():
            pltpu.sync_copy(den_sh.at[pl.ds(s * 8, 8)],
                            den0_hbm.at[pl.ds(s * 8, 8)])

    @pl.when(c == 1)
    def _():
        pltpu.sync_copy(acc_sh.at[pl.ds(s * DST_PER_TILE, DST_PER_TILE)],
                        out1_hbm.at[pl.ds(s * DST_PER_TILE, DST_PER_TILE)])

        @pl.when(s < DR // 8)
        def _():
            pltpu.sync_copy(den_sh.at[pl.ds(s * 8, 8)],
                            den1_hbm.at[pl.ds(s * 8, 8)])


_push = pl.kernel(
    _push_body,
    out_type=(jax.ShapeDtypeStruct((NP, F), jnp.float32),
              jax.ShapeDtypeStruct((NP, F), jnp.float32),
              jax.ShapeDtypeStruct((DR, F), jnp.float32),
              jax.ShapeDtypeStruct((DR, F), jnp.float32)),
    mesh=_mesh,
    compiler_params=pltpu.CompilerParams(needs_layout_passes=False),
    scratch_types=[
        pltpu.VMEM((2, B, F), jnp.float32),
        pltpu.VMEM((2, CR), jnp.float32),
        pltpu.VMEM((2, CR), jnp.int32),
        pltpu.VMEM((2, CR), jnp.int32),
        pltpu.VMEM((2, CR, F), jnp.float32),
        pltpu.VMEM((DR, F), jnp.float32),
        pltpu.VMEM((DR,), jnp.int32),
        pltpu.VMEM_SHARED((NP, F), jnp.float32),
        pltpu.VMEM_SHARED((DR, F), jnp.float32),
        pltpu.SemaphoreType.DMA,
        pltpu.SemaphoreType.DMA,
        pltpu.SemaphoreType.DMA,
        pltpu.SemaphoreType.DMA,
        pltpu.SemaphoreType.DMA,
    ],
)


def _up_body(p0_hbm, p1_hbm, d0_hbm, d1_hbm, sel_hbm, out_hbm,
             idx_v, r0_v, r1_v, den0_v, den1_v, o_v, sem, dsem):
    c = lax.axis_index("c")
    s = lax.axis_index("s")
    wid = c * NS + s
    base = wid * UP_PER_TILE
    d0c = pltpu.make_async_copy(d0_hbm, den0_v, dsem)
    d1c = pltpu.make_async_copy(d1_hbm, den1_v, dsem)
    d0c.start()
    d1c.start()
    pltpu.sync_copy(sel_hbm.at[pl.ds(base, UP_PER_TILE)], idx_v)
    g0 = pltpu.async_copy(p0_hbm.at[idx_v], r0_v, sem)
    g1 = pltpu.async_copy(p1_hbm.at[idx_v], r1_v, sem)
    d0c.wait()
    d1c.wait()
    g0.wait()
    g1.wait()

    def grp(g, carry):
        selvec = idx_v[pl.ds(g * 16, 16)]
        ihi = lax.shift_right_logical(selvec, 7)
        ilo = lax.bitwise_and(selvec, 127)
        den = (plsc.load_gather(den0_v, [ihi, ilo])
               + plsc.load_gather(den1_v, [ihi, ilo])
               + jnp.float32(0.001))
        scale = jnp.where(den == jnp.float32(0.0),
                          jnp.float32(0.0), jnp.float32(1.0) / den)
        for jj in range(16):
            j = g * 16 + jj
            sj = scale[jj]
            for v in range(NV):
                sl = pl.ds(v * 16, 16)
                o_v[j, sl] = (r0_v[j, sl] + r1_v[j, sl]) * sj
        return carry
    lax.fori_loop(0, UP_PER_TILE // 16, grp, 0)
    pltpu.sync_copy(o_v, out_hbm.at[pl.ds(base, UP_PER_TILE)])


_up = pl.kernel(
    _up_body,
    out_type=jax.ShapeDtypeStruct((NUP_P, F), jnp.float32),
    mesh=_mesh,
    compiler_params=pltpu.CompilerParams(needs_layout_passes=False),
    scratch_types=[
        pltpu.VMEM((UP_PER_TILE,), jnp.int32),
        pltpu.VMEM((UP_PER_TILE, F), jnp.float32),
        pltpu.VMEM((UP_PER_TILE, F), jnp.float32),
        pltpu.VMEM((DR, F), jnp.float32),
        pltpu.VMEM((DR, F), jnp.float32),
        pltpu.VMEM((UP_PER_TILE, F), jnp.float32),
        pltpu.SemaphoreType.DMA,
        pltpu.SemaphoreType.DMA,
    ],
)


@jax.jit
def kernel(features, nidx_down, weights_down, sel_idx_up):
    nidx_flat = nidx_down.reshape(-1)
    w_flat = weights_down.reshape(-1)
    sel_p = jnp.pad(sel_idx_up[:, 0], (0, NUP_P - N_UP))
    out0, out1, den0, den1 = _push(features, nidx_flat, w_flat)
    res = _up(out0, out1, den0, den1, sel_p)
    return res[:N_UP]


# speedup vs baseline: 1.4708x; 1.4708x over previous
import jax
import jax.numpy as jnp
from jax import lax
from jax.experimental import pallas as pl
from jax.experimental.pallas import tpu as pltpu
from jax.experimental.pallas import tpu_sc as plsc

N, K, F, N_UP = 10000, 32, 128, 2500
NC, NS = 2, 16
NW = NC * NS
NP = 10240
ROWS_PER_TILE = NP // NW
B = 8
NBLK = ROWS_PER_TILE // B
CR = B * K
CH = CR // 128
PG = K // 128 * 0 + (B * K) // 128
DST_PER_TILE = NP // NS
DR = NP // F
NUP_P = 2560
UP_PER_TILE = NUP_P // NW
NV = F // 16

_mesh = plsc.VectorSubcoreMesh(
    core_axis_name="c", subcore_axis_name="s", num_cores=NC, num_subcores=NS)


def _push_body(feat_hbm, packed_hbm,
               out0_hbm, out1_hbm, den0_hbm, den1_hbm,
               feat_v, pw_v, idxs_v, contrib_v, den_v, idxid_v,
               acc_sh, den_sh,
               isem0, isem1, ssem0, ssem1, zsem):
    c = lax.axis_index("c")
    s = lax.axis_index("s")
    wid = c * NS + s
    zvec = jnp.zeros((16,), jnp.float32)
    isems = (isem0, isem1)
    ssems = (ssem0, ssem1)
    nblk = (jnp.minimum(ROWS_PER_TILE, N - wid * ROWS_PER_TILE)
            + (B - 1)) // B

    def in_copies(b, buf):
        base = wid * ROWS_PER_TILE + b * B
        grp = wid * (ROWS_PER_TILE * K // 128) + b * PG
        return (
            pltpu.make_async_copy(feat_hbm.at[pl.ds(base, B)],
                                  feat_v.at[buf], isems[buf]),
            pltpu.make_async_copy(packed_hbm.at[pl.ds(grp, PG)],
                                  pw_v.at[buf], isems[buf]),
        )

    def fire_inputs(b, buf):
        for d in in_copies(b, buf):
            d.start()

    def drain_inputs(b, buf):
        for d in in_copies(b, buf):
            d.wait()

    def scat_copy(hb):
        return pltpu.make_async_copy(
            contrib_v.at[hb], acc_sh.at[idxs_v.at[hb]], ssems[hb])

    fire_inputs(0, 0)

    def zrow(j, carry):
        for v in range(NV):
            contrib_v[1, j, pl.ds(v * 16, 16)] = zvec
        return carry
    lax.fori_loop(0, 128, zrow, 0)

    zcopies = [
        pltpu.make_async_copy(
            contrib_v.at[1],
            acc_sh.at[pl.ds(s * DST_PER_TILE + j * 128, 128)], zsem)
        for j in range(DST_PER_TILE // 128)
    ]
    zdcopy = pltpu.make_async_copy(contrib_v.at[1, pl.ds(0, 8)],
                                   den_sh.at[pl.ds(s * 8, 8)], zsem)
    for d in zcopies:
        d.start()

    @pl.when(s < DR // 8)
    def _():
        zdcopy.start()

    def zden(j, carry):
        for v in range(NV):
            den_v[j, pl.ds(v * 16, 16)] = zvec
        return carry
    lax.fori_loop(0, DR, zden, 0)

    iota16 = lax.iota(jnp.int32, 16)
    for g in range(DR // 16):
        idxid_v[pl.ds(g * 16, 16)] = iota16 + g * 16

    for d in zcopies:
        d.wait()

    @pl.when(s < DR // 8)
    def _():
        zdcopy.wait()
    plsc.subcore_barrier()

    def pair(p, carry):
        for par in range(2):
            b = 2 * p + par
            drain_inputs(b, par)

            @pl.when(b + 1 < nblk)
            def _():
                fire_inputs(b + 1, 1 - par)

            for hb in range(2):
                @pl.when(b > 0)
                def _():
                    scat_copy(hb).wait()

                def row(i, carry2):
                    gi = hb * 4 + i
                    f = [feat_v[par, gi, pl.ds(v * 16, 16)]
                         for v in range(NV)]
                    off = K * i
                    m0 = i * K
                    for kh in range(K // 16):
                        wvec = plsc.bitcast(
                            pw_v[par, hb, 1, pl.ds(off + kh * 16, 16)],
                            jnp.float32)
                        for kk in range(16):
                            k = kh * 16 + kk
                            wk = wvec[kk]
                            for v in range(NV):
                                contrib_v[hb, m0 + k,
                                          pl.ds(v * 16, 16)] = wk * f[v]
                    return carry2
                lax.fori_loop(0, B // 2, row, 0)

                for gg in range(8):
                    ivec = pw_v[par, hb, 0, pl.ds(gg * 16, 16)]
                    wvec = plsc.bitcast(
                        pw_v[par, hb, 1, pl.ds(gg * 16, 16)], jnp.float32)
                    idxs_v[hb, pl.ds(gg * 16, 16)] = ivec
                    plsc.addupdate_scatter(
                        den_v,
                        [lax.shift_right_logical(ivec, 7),
                         lax.bitwise_and(ivec, 127)],
                        wvec)
                scat_copy(hb).start(add=True)
        return carry
    lax.fori_loop(0, nblk // 2, pair, 0)

    for hb in range(2):
        scat_copy(hb).wait()

    pltpu.sync_copy(den_v, den_sh.at[idxid_v], add=True)
    plsc.subcore_barrier()

    @pl.when(c == 0)
    def _():
        pltpu.sync_copy(acc_sh.at[pl.ds(s * DST_PER_TILE, DST_PER_TILE)],
                        out0_hbm.at[pl.ds(s * DST_PER_TILE, DST_PER_TILE)])

        @pl.when(s < DR // 8)
        def _():
            pltpu.sync_copy(den_sh.at[pl.ds(s * 8, 8)],
                            den0_hbm.at[pl.ds(s * 8, 8)])

    @pl.when(c == 1)
    def _():
        pltpu.sync_copy(acc_sh.at[pl.ds(s * DST_PER_TILE, DST_PER_TILE)],
                        out1_hbm.at[pl.ds(s * DST_PER_TILE, DST_PER_TILE)])

        @pl.when(s < DR // 8)
        def _():
            pltpu.sync_copy(den_sh.at[pl.ds(s * 8, 8)],
                            den1_hbm.at[pl.ds(s * 8, 8)])


_push = pl.kernel(
    _push_body,
    out_type=(jax.ShapeDtypeStruct((NP, F), jnp.float32),
              jax.ShapeDtypeStruct((NP, F), jnp.float32),
              jax.ShapeDtypeStruct((DR, F), jnp.float32),
              jax.ShapeDtypeStruct((DR, F), jnp.float32)),
    mesh=_mesh,
    compiler_params=pltpu.CompilerParams(needs_layout_passes=False),
    scratch_types=[
        pltpu.VMEM((2, B, F), jnp.float32),
        pltpu.VMEM((2, PG, 2, 128), jnp.int32),
        pltpu.VMEM((2, 128), jnp.int32),
        pltpu.VMEM((2, 128, F), jnp.float32),
        pltpu.VMEM((DR, F), jnp.float32),
        pltpu.VMEM((DR,), jnp.int32),
        pltpu.VMEM_SHARED((NP, F), jnp.float32),
        pltpu.VMEM_SHARED((DR, F), jnp.float32),
        pltpu.SemaphoreType.DMA,
        pltpu.SemaphoreType.DMA,
        pltpu.SemaphoreType.DMA,
        pltpu.SemaphoreType.DMA,
        pltpu.SemaphoreType.DMA,
    ],
)


def _up_body(p0_hbm, p1_hbm, d0_hbm, d1_hbm, sel_hbm, out_hbm,
             idx_v, r0_v, r1_v, den0_v, den1_v, o_v, sem, dsem):
    c = lax.axis_index("c")
    s = lax.axis_index("s")
    wid = c * NS + s
    base = wid * UP_PER_TILE
    d0c = pltpu.make_async_copy(d0_hbm, den0_v, dsem)
    d1c = pltpu.make_async_copy(d1_hbm, den1_v, dsem)
    d0c.start()
    d1c.start()
    pltpu.sync_copy(sel_hbm.at[pl.ds(base, UP_PER_TILE)], idx_v)
    g0 = pltpu.async_copy(p0_hbm.at[idx_v], r0_v, sem)
    g1 = pltpu.async_copy(p1_hbm.at[idx_v], r1_v, sem)
    d0c.wait()
    d1c.wait()
    g0.wait()
    g1.wait()

    def grp(g, carry):
        selvec = idx_v[pl.ds(g * 16, 16)]
        ihi = lax.shift_right_logical(selvec, 7)
        ilo = lax.bitwise_and(selvec, 127)
        den = (plsc.load_gather(den0_v, [ihi, ilo])
               + plsc.load_gather(den1_v, [ihi, ilo])
               + jnp.float32(0.001))
        scale = jnp.where(den == jnp.float32(0.0),
                          jnp.float32(0.0), jnp.float32(1.0) / den)
        for jj in range(16):
            j = g * 16 + jj
            sj = scale[jj]
            for v in range(NV):
                sl = pl.ds(v * 16, 16)
                o_v[j, sl] = (r0_v[j, sl] + r1_v[j, sl]) * sj
        return carry
    lax.fori_loop(0, UP_PER_TILE // 16, grp, 0)
    pltpu.sync_copy(o_v, out_hbm.at[pl.ds(base, UP_PER_TILE)])


_up = pl.kernel(
    _up_body,
    out_type=jax.ShapeDtypeStruct((NUP_P, F), jnp.float32),
    mesh=_mesh,
    compiler_params=pltpu.CompilerParams(needs_layout_passes=False),
    scratch_types=[
        pltpu.VMEM((UP_PER_TILE,), jnp.int32),
        pltpu.VMEM((UP_PER_TILE, F), jnp.float32),
        pltpu.VMEM((UP_PER_TILE, F), jnp.float32),
        pltpu.VMEM((DR, F), jnp.float32),
        pltpu.VMEM((DR, F), jnp.float32),
        pltpu.VMEM((UP_PER_TILE, F), jnp.float32),
        pltpu.SemaphoreType.DMA,
        pltpu.SemaphoreType.DMA,
    ],
)


@jax.jit
def kernel(features, nidx_down, weights_down, sel_idx_up):
    nidx_g = nidx_down.reshape(-1, 128)
    w_g = lax.bitcast_convert_type(weights_down, jnp.int32).reshape(-1, 128)
    packed = jnp.stack([nidx_g, w_g], axis=1)
    sel_p = jnp.pad(sel_idx_up[:, 0], (0, NUP_P - N_UP))
    out0, out1, den0, den1 = _push(features, packed)
    res = _up(out0, out1, den0, den1, sel_p)
    return res[:N_UP]


# speedup vs baseline: 1.4776x; 1.0046x over previous
import jax
import jax.numpy as jnp
from jax import lax
from jax.experimental import pallas as pl
from jax.experimental.pallas import tpu as pltpu
from jax.experimental.pallas import tpu_sc as plsc

N, K, F, N_UP = 10000, 32, 128, 2500
NC, NS = 2, 16
NW = NC * NS
NP = 10240
ROWS_PER_TILE = NP // NW
B = 8
NBLK = ROWS_PER_TILE // B
CR = B * K
CH = CR // 128
PG = K // 128 * 0 + (B * K) // 128
DST_PER_TILE = NP // NS
DR = NP // F
NUP_P = 2560
UP_PER_TILE = NUP_P // NW
NV = F // 16

_mesh = plsc.VectorSubcoreMesh(
    core_axis_name="c", subcore_axis_name="s", num_cores=NC, num_subcores=NS)


def _push_body(feat_hbm, packed_hbm,
               out0_hbm, out1_hbm, den0_hbm, den1_hbm,
               feat_v, pw_v, idxs_v, contrib_v, den_v, idxid_v, dummy_v,
               acc_sh, den_sh,
               isem0, isem1, ssem0, ssem1, zsem):
    c = lax.axis_index("c")
    s = lax.axis_index("s")
    wid = c * NS + s
    zvec = jnp.zeros((16,), jnp.float32)
    isems = (isem0, isem1)
    ssems = (ssem0, ssem1)
    nblk = (jnp.minimum(ROWS_PER_TILE, N - wid * ROWS_PER_TILE)
            + (B - 1)) // B

    def in_copies(b, buf):
        base = wid * ROWS_PER_TILE + b * B
        grp = wid * (ROWS_PER_TILE * K // 128) + b * PG
        return (
            pltpu.make_async_copy(feat_hbm.at[pl.ds(base, B)],
                                  feat_v.at[buf], isems[buf]),
            pltpu.make_async_copy(packed_hbm.at[pl.ds(grp, PG)],
                                  pw_v.at[buf], isems[buf]),
        )

    def fire_inputs(b, buf):
        for d in in_copies(b, buf):
            d.start()

    def drain_inputs(b, buf):
        pltpu.make_async_copy(packed_hbm.at[pl.ds(0, 6)], dummy_v,
                              isems[buf]).wait()

    def scat_copy(hb):
        return pltpu.make_async_copy(
            contrib_v.at[hb], acc_sh.at[idxs_v.at[hb]], ssems[hb])

    fire_inputs(0, 0)

    def zrow(j, carry):
        for v in range(NV):
            contrib_v[1, j, pl.ds(v * 16, 16)] = zvec
        return carry
    lax.fori_loop(0, 128, zrow, 0)

    zcopies = [
        pltpu.make_async_copy(
            contrib_v.at[1],
            acc_sh.at[pl.ds(s * DST_PER_TILE + j * 128, 128)], zsem)
        for j in range(DST_PER_TILE // 128)
    ]
    zdcopy = pltpu.make_async_copy(contrib_v.at[1, pl.ds(0, 8)],
                                   den_sh.at[pl.ds(s * 8, 8)], zsem)
    for d in zcopies:
        d.start()

    @pl.when(s < DR // 8)
    def _():
        zdcopy.start()

    def zden(j, carry):
        for v in range(NV):
            den_v[j, pl.ds(v * 16, 16)] = zvec
        return carry
    lax.fori_loop(0, DR, zden, 0)

    iota16 = lax.iota(jnp.int32, 16)
    for g in range(DR // 16):
        idxid_v[pl.ds(g * 16, 16)] = iota16 + g * 16

    for d in zcopies:
        d.wait()

    @pl.when(s < DR // 8)
    def _():
        zdcopy.wait()
    plsc.subcore_barrier()

    def pair(p, carry):
        for par in range(2):
            b = 2 * p + par
            drain_inputs(b, par)

            @pl.when(b + 1 < nblk)
            def _():
                fire_inputs(b + 1, 1 - par)

            for hb in range(2):
                @pl.when(b > 0)
                def _():
                    scat_copy(hb).wait()

                def row(i, carry2):
                    gi = hb * 4 + i
                    f = [feat_v[par, gi, pl.ds(v * 16, 16)]
                         for v in range(NV)]
                    off = K * i
                    m0 = i * K
                    for kh in range(K // 16):
                        wvec = plsc.bitcast(
                            pw_v[par, hb, 1, pl.ds(off + kh * 16, 16)],
                            jnp.float32)
                        for kk in range(16):
                            k = kh * 16 + kk
                            wk = wvec[kk]
                            for v in range(NV):
                                contrib_v[hb, m0 + k,
                                          pl.ds(v * 16, 16)] = wk * f[v]
                    return carry2
                lax.fori_loop(0, B // 2, row, 0)

                for gg in range(8):
                    ivec = pw_v[par, hb, 0, pl.ds(gg * 16, 16)]
                    wvec = plsc.bitcast(
                        pw_v[par, hb, 1, pl.ds(gg * 16, 16)], jnp.float32)
                    idxs_v[hb, pl.ds(gg * 16, 16)] = ivec
                    plsc.addupdate_scatter(
                        den_v,
                        [lax.shift_right_logical(ivec, 7),
                         lax.bitwise_and(ivec, 127)],
                        wvec)
                scat_copy(hb).start(add=True)
        return carry
    lax.fori_loop(0, nblk // 2, pair, 0)

    for hb in range(2):
        scat_copy(hb).wait()

    pltpu.sync_copy(den_v, den_sh.at[idxid_v], add=True)
    plsc.subcore_barrier()

    @pl.when(c == 0)
    def _():
        pltpu.sync_copy(acc_sh.at[pl.ds(s * DST_PER_TILE, DST_PER_TILE)],
                        out0_hbm.at[pl.ds(s * DST_PER_TILE, DST_PER_TILE)])

        @pl.when(s < DR // 8)
        def _():
            pltpu.sync_copy(den_sh.at[pl.ds(s * 8, 8)],
                            den0_hbm.at[pl.ds(s * 8, 8)])

    @pl.when(c == 1)
    def _():
        pltpu.sync_copy(acc_sh.at[pl.ds(s * DST_PER_TILE, DST_PER_TILE)],
                        out1_hbm.at[pl.ds(s * DST_PER_TILE, DST_PER_TILE)])

        @pl.when(s < DR // 8)
        def _():
            pltpu.sync_copy(den_sh.at[pl.ds(s * 8, 8)],
                            den1_hbm.at[pl.ds(s * 8, 8)])


_push = pl.kernel(
    _push_body,
    out_type=(jax.ShapeDtypeStruct((NP, F), jnp.float32),
              jax.ShapeDtypeStruct((NP, F), jnp.float32),
              jax.ShapeDtypeStruct((DR, F), jnp.float32),
              jax.ShapeDtypeStruct((DR, F), jnp.float32)),
    mesh=_mesh,
    compiler_params=pltpu.CompilerParams(needs_layout_passes=False),
    scratch_types=[
        pltpu.VMEM((2, B, F), jnp.float32),
        pltpu.VMEM((2, PG, 2, 128), jnp.int32),
        pltpu.VMEM((2, 128), jnp.int32),
        pltpu.VMEM((2, 128, F), jnp.float32),
        pltpu.VMEM((DR, F), jnp.float32),
        pltpu.VMEM((DR,), jnp.int32),
        pltpu.VMEM((6, 2, 128), jnp.int32),
        pltpu.VMEM_SHARED((NP, F), jnp.float32),
        pltpu.VMEM_SHARED((DR, F), jnp.float32),
        pltpu.SemaphoreType.DMA,
        pltpu.SemaphoreType.DMA,
        pltpu.SemaphoreType.DMA,
        pltpu.SemaphoreType.DMA,
        pltpu.SemaphoreType.DMA,
    ],
)


def _up_body(p0_hbm, p1_hbm, d0_hbm, d1_hbm, sel_hbm, out_hbm,
             idx_v, r0_v, r1_v, den0_v, den1_v, o_v, sem, dsem):
    c = lax.axis_index("c")
    s = lax.axis_index("s")
    wid = c * NS + s
    base = wid * UP_PER_TILE
    d0c = pltpu.make_async_copy(d0_hbm, den0_v, dsem)
    d1c = pltpu.make_async_copy(d1_hbm, den1_v, dsem)
    d0c.start()
    d1c.start()
    pltpu.sync_copy(sel_hbm.at[pl.ds(base, UP_PER_TILE)], idx_v)
    g0 = pltpu.async_copy(p0_hbm.at[idx_v], r0_v, sem)
    g1 = pltpu.async_copy(p1_hbm.at[idx_v], r1_v, sem)
    d0c.wait()
    d1c.wait()
    g0.wait()
    g1.wait()

    def grp(g, carry):
        selvec = idx_v[pl.ds(g * 16, 16)]
        ihi = lax.shift_right_logical(selvec, 7)
        ilo = lax.bitwise_and(selvec, 127)
        den = (plsc.load_gather(den0_v, [ihi, ilo])
               + plsc.load_gather(den1_v, [ihi, ilo])
               + jnp.float32(0.001))
        scale = jnp.where(den == jnp.float32(0.0),
                          jnp.float32(0.0), jnp.float32(1.0) / den)
        for jj in range(16):
            j = g * 16 + jj
            sj = scale[jj]
            for v in range(NV):
                sl = pl.ds(v * 16, 16)
                o_v[j, sl] = (r0_v[j, sl] + r1_v[j, sl]) * sj
        return carry
    lax.fori_loop(0, UP_PER_TILE // 16, grp, 0)
    pltpu.sync_copy(o_v, out_hbm.at[pl.ds(base, UP_PER_TILE)])


_up = pl.kernel(
    _up_body,
    out_type=jax.ShapeDtypeStruct((NUP_P, F), jnp.float32),
    mesh=_mesh,
    compiler_params=pltpu.CompilerParams(needs_layout_passes=False),
    scratch_types=[
        pltpu.VMEM((UP_PER_TILE,), jnp.int32),
        pltpu.VMEM((UP_PER_TILE, F), jnp.float32),
        pltpu.VMEM((UP_PER_TILE, F), jnp.float32),
        pltpu.VMEM((DR, F), jnp.float32),
        pltpu.VMEM((DR, F), jnp.float32),
        pltpu.VMEM((UP_PER_TILE, F), jnp.float32),
        pltpu.SemaphoreType.DMA,
        pltpu.SemaphoreType.DMA,
    ],
)


@jax.jit
def kernel(features, nidx_down, weights_down, sel_idx_up):
    nidx_g = nidx_down.reshape(-1, 128)
    w_g = lax.bitcast_convert_type(weights_down, jnp.int32).reshape(-1, 128)
    packed = jnp.stack([nidx_g, w_g], axis=1)
    sel_p = jnp.pad(sel_idx_up[:, 0], (0, NUP_P - N_UP))
    out0, out1, den0, den1 = _push(features, packed)
    res = _up(out0, out1, den0, den1, sel_p)
    return res[:N_UP]
